# baseline (speedup 1.0000x reference)
import dataclasses
import functools

import jax
import jax.numpy as jnp
from jax import lax
from jax.experimental import pallas as pl
from jax.experimental.pallas import tpu as pltpu
from jax.experimental.pallas import tpu_sc as plsc

B, C = 16384, 128
NC, NS, L = 2, 16, 16
NW = NC * NS
ROWS_W = B // NW
GROUPS = ROWS_W // L

K1 = (C - 1) * C / 2.0
K2 = (C - 1) * C * (2 * C - 1) / 6.0

_LOG_COEF = (7.0376836292e-2, -1.1514610310e-1, 1.1676998740e-1,
             -1.2420140846e-1, 1.4249322787e-1, -1.6668057665e-1,
             2.0000714765e-1, -2.4999993993e-1, 3.3333331174e-1)


def _logf(y):
    yi = lax.bitcast_convert_type(y, jnp.int32)
    ex = lax.shift_right_arithmetic(yi, 23) - 127
    mi = jnp.bitwise_or(jnp.bitwise_and(yi, 0x007FFFFF), 0x3F800000)
    m = lax.bitcast_convert_type(mi, jnp.float32)
    big = m > jnp.float32(1.41421356)
    m = jnp.where(big, m * jnp.float32(0.5), m)
    ex = jnp.where(big, ex + 1, ex)
    e = ex.astype(jnp.float32)
    z = m - jnp.float32(1.0)
    zz = z * z
    p = jnp.float32(_LOG_COEF[0])
    for c in _LOG_COEF[1:]:
        p = p * z + jnp.float32(c)
    r = z * zz * p
    r = r + e * jnp.float32(-2.12194440e-4)
    r = r - jnp.float32(0.5) * zz
    r = r + z
    r = r + e * jnp.float32(0.693359375)
    return r


_mesh = plsc.VectorSubcoreMesh(core_axis_name="core", subcore_axis_name="subcore")

_cp = pltpu.CompilerParams()
if "needs_layout_passes" in pltpu.CompilerParams.__dataclass_fields__:
    _cp = dataclasses.replace(_cp, needs_layout_passes=False)


@functools.partial(
    pl.kernel,
    compiler_params=_cp,
    out_type=jax.ShapeDtypeStruct((NW, L), jnp.float32),
    mesh=_mesh,
    scratch_types=[
        pltpu.VMEM((ROWS_W, C), jnp.float32),
        pltpu.VMEM((GROUPS, L), jnp.int32),
        pltpu.VMEM((C,), jnp.float32),
        pltpu.VMEM((L,), jnp.float32),
    ],
)
def _sc_loss(x_hbm, t_hbm, w_hbm, out_hbm, x_v, t_v, w_v, acc_v):
    wid = lax.axis_index("subcore") * NC + lax.axis_index("core")
    pltpu.sync_copy(x_hbm.at[pl.ds(wid * ROWS_W, ROWS_W)], x_v)
    pltpu.sync_copy(t_hbm.at[pl.ds(wid * GROUPS, GROUPS)], t_v)
    pltpu.sync_copy(w_hbm, w_v)

    iota = lax.broadcasted_iota(jnp.int32, (L,), 0)

    def group_body(g, acc):
        t = t_v[g]
        tf = t.astype(jnp.float32)
        rows = g * L + iota
        s_t = jnp.float32(C) * tf * tf - jnp.float32(2.0 * K1) * tf + jnp.float32(K2)
        scale = jnp.float32(C - 1) / s_t

        z_acc = jnp.zeros((L,), jnp.float32)
        for j in range(C):
            col = jnp.full((L,), j, jnp.int32)
            xv = plsc.load_gather(x_v, [rows, col])
            z_acc = z_acc + jnp.exp(xv)
        inv_z = jnp.float32(1.0) / z_acc
        log_z = _logf(z_acc)

        xt = plsc.load_gather(x_v, [rows, t])
        wt = plsc.load_gather(w_v, [t])
        acc = acc + wt * (xt - log_z)

        nt = jnp.zeros((L,), jnp.float32)
        for j in range(C):
            col = jnp.full((L,), j, jnp.int32)
            xv = plsc.load_gather(x_v, [rows, col])
            s = jnp.exp(xv) * inv_z
            lg = _logf(jnp.float32(1.0) - s)
            dj = jnp.float32(j) - tf
            wj = plsc.load_gather(w_v, [col])
            nt = nt + (dj * dj) * lg * wj
        return acc + scale * nt

    acc = lax.fori_loop(0, GROUPS, group_body, jnp.zeros((L,), jnp.float32))
    acc_v[...] = acc * jnp.float32(-1.0 / 1000.0)
    pltpu.sync_copy(acc_v, out_hbm.at[wid])


def _tc_finish(partials):
    def body(p_ref, o_ref):
        o_ref[...] = jnp.sum(p_ref[...])[None, None]

    return pl.pallas_call(
        body, out_shape=jax.ShapeDtypeStruct((1, 1), jnp.float32))(partials)


def kernel(inputs, target, weight):
    t2 = target.reshape(B // L, L)
    partials = _sc_loss(inputs, t2, weight)
    return _tc_finish(partials)[0, 0]
